# baseline (speedup 1.0000x reference)
import jax
import jax.numpy as jnp
from jax import lax
from jax.experimental import pallas as pl
from jax.experimental.pallas import tpu as pltpu
from jax.experimental.pallas import tpu_sc as plsc

B = 4096
L = 200
D = 64
NC = 2
NS = 16
NW = NC * NS
CB = 128
NCB = B // CB
NIT = (L * NCB) // NW
NBUF = 4
PF = 2


def _embt_body(xt_hbm, tok_hbm, pos_hbm, out_hbm, idx_v, pos_v, gbuf, obuf,
               *sems):
  gsem = sems[:NBUF]
  wsem = sems[NBUF:]
  wid = lax.axis_index("s") * NC + lax.axis_index("c")
  base_it = wid * NIT

  pltpu.sync_copy(xt_hbm.at[pl.ds(base_it * CB, NIT * CB)], idx_v)
  pltpu.sync_copy(pos_hbm, pos_v)

  def lc(i):
    g = base_it + i
    return lax.div(g, NCB), lax.rem(g, NCB)

  def issue_gather(i, b):
    pltpu.async_copy(tok_hbm.at[idx_v.at[pl.ds(i * CB, CB)]], gbuf.at[b],
                     gsem[b])

  def wait_gather(i, b):
    pltpu.make_async_copy(tok_hbm.at[idx_v.at[pl.ds(i * CB, CB)]], gbuf.at[b],
                          gsem[b]).wait()

  def issue_write(i, b):
    l, c = lc(i)
    for dh in range(8):
      pltpu.async_copy(obuf.at[b, dh], out_hbm.at[l, dh, c], wsem[b])

  def wait_write(i, b):
    l, c = lc(i)
    for dh in range(8):
      pltpu.make_async_copy(obuf.at[b, dh], out_hbm.at[l, dh, c],
                            wsem[b]).wait()

  def transpose_add(i, b):
    l, _ = lc(i)
    gb = gbuf.at[b]
    ob = obuf.at[b]
    lvec = jnp.full((16,), l, jnp.int32)

    @plsc.parallel_loop(0, D, unroll=2)
    def _(d):
      dh = lax.div(d, 8)
      dl = lax.rem(d, 8)
      dvec = jnp.full((16,), d, jnp.int32)
      ps = plsc.load_gather(pos_v, [lvec, dvec])
      for k in range(8):
        rows = lax.iota(jnp.int32, 16) + (16 * k)
        val = plsc.load_gather(gb, [rows, dvec]) + ps
        ob[dh, dl, pl.ds(16 * k, 16)] = val

  def do_item(i, b, prefetch, drain):
    wait_gather(i, b)
    transpose_add(i, b)
    issue_write(i, b)
    if prefetch:
      tgt = i + PF
      bp = (b + PF) % NBUF
      if drain:
        wait_write(tgt - NBUF, bp)
      issue_gather(tgt, bp)

  for b in range(PF):
    issue_gather(jnp.int32(b), b)

  for b in range(NBUF):
    do_item(jnp.int32(b), b, prefetch=True, drain=(b >= PF))

  def outer(step, _):
    for b in range(NBUF):
      i = step * NBUF + b
      do_item(i, b, prefetch=True, drain=True)
    return 0

  lax.fori_loop(1, NIT // NBUF - 1, outer, 0)

  base = jnp.int32(NIT - NBUF)
  for b in range(NBUF):
    do_item(base + b, b, prefetch=(b < PF), drain=True)

  for b in range(NBUF):
    wait_write(base + b, b)


def _embt(xt1, token_table, pos_table):
  mesh = plsc.VectorSubcoreMesh(core_axis_name="c", subcore_axis_name="s")
  scratch = [
      pltpu.VMEM((NIT * CB,), jnp.int32),
      pltpu.VMEM((L, D), jnp.float32),
      pltpu.VMEM((NBUF, CB, D), jnp.float32),
      pltpu.VMEM((NBUF, 8, 8, CB), jnp.float32),
  ] + [pltpu.SemaphoreType.DMA] * (2 * NBUF)
  f = pl.kernel(
      _embt_body,
      out_type=jax.ShapeDtypeStruct((L, 8, NCB, 8, CB), jnp.float32),
      mesh=mesh,
      scratch_types=scratch,
      compiler_params=pltpu.CompilerParams(
          use_tc_tiling_on_sc=False, needs_layout_passes=False),
  )
  return f(xt1, token_table, pos_table)


def kernel(x, token_table, pos_table):
  b, l = x.shape
  d = token_table.shape[1]
  assert (b, l, d) == (B, L, D)
  xt1 = x.astype(jnp.int32).T.reshape(-1)
  out_phys = _embt(xt1, token_table, pos_table)
  return out_phys.transpose((2, 4, 0, 1, 3)).reshape(B, L, D)


# speedup vs baseline: 2.9255x; 2.9255x over previous
import jax
import jax.numpy as jnp
from jax import lax
from jax.experimental import pallas as pl
from jax.experimental.pallas import tpu as pltpu
from jax.experimental.pallas import tpu_sc as plsc

B = 4096
L = 200
D = 64
NC = 2
NS = 16
NW = NC * NS
CB = 128
NCB = B // CB
NIT = (L * NCB) // NW
NBUF = 4
PF = 2
OBW = 136


def _embt_body(xt_hbm, tok_hbm, pos_hbm, out_hbm, idx_v, pos_v, gbuf, obuf,
               *sems):
  gsem = sems[:NBUF]
  wsem = sems[NBUF:]
  wid = lax.axis_index("s") * NC + lax.axis_index("c")
  base_it = wid * NIT

  pltpu.sync_copy(xt_hbm.at[pl.ds(base_it * CB, NIT * CB)], idx_v)
  pltpu.sync_copy(pos_hbm, pos_v)

  def lc(i):
    g = base_it + i
    return lax.div(g, NCB), lax.rem(g, NCB)

  def issue_gather(i, b):
    pltpu.async_copy(tok_hbm.at[idx_v.at[pl.ds(i * CB, CB)]], gbuf.at[b],
                     gsem[b])

  def wait_gather(i, b):
    pltpu.make_async_copy(tok_hbm.at[idx_v.at[pl.ds(i * CB, CB)]], gbuf.at[b],
                          gsem[b]).wait()

  def ob_view(b, dh):
    return obuf.at[b, pl.ds(dh * 8, 8), pl.ds(0, CB)]

  def issue_write(i, b):
    l, c = lc(i)
    for dh in range(8):
      pltpu.async_copy(ob_view(b, dh), out_hbm.at[l, dh, c], wsem[b])

  def wait_write(i, b):
    l, c = lc(i)
    for dh in range(8):
      pltpu.make_async_copy(ob_view(b, dh), out_hbm.at[l, dh, c],
                            wsem[b]).wait()

  def transpose_add(i, b):
    l, _ = lc(i)
    gb = gbuf.at[b]
    ob = obuf.at[b]
    pos_rows = [pos_v[l, pl.ds(16 * j, 16)] for j in range(D // 16)]
    dvecs = [lax.iota(jnp.int32, 16) + (16 * j) for j in range(D // 16)]

    @plsc.parallel_loop(0, CB, unroll=2)
    def _(bb):
      bvec = jnp.full((16,), bb, jnp.int32)
      for j in range(D // 16):
        val = gb[bb, pl.ds(16 * j, 16)] + pos_rows[j]
        plsc.store_scatter(ob, [dvecs[j], bvec], val)

  def do_item(i, b, prefetch, drain):
    wait_gather(i, b)
    transpose_add(i, b)
    issue_write(i, b)
    if prefetch:
      tgt = i + PF
      bp = (b + PF) % NBUF
      if drain:
        wait_write(tgt - NBUF, bp)
      issue_gather(tgt, bp)

  for b in range(PF):
    issue_gather(jnp.int32(b), b)

  for b in range(NBUF):
    do_item(jnp.int32(b), b, prefetch=True, drain=(b >= PF))

  def outer(step, _):
    for b in range(NBUF):
      i = step * NBUF + b
      do_item(i, b, prefetch=True, drain=True)
    return 0

  lax.fori_loop(1, NIT // NBUF - 1, outer, 0)

  base = jnp.int32(NIT - NBUF)
  for b in range(NBUF):
    do_item(base + b, b, prefetch=(b < PF), drain=True)

  for b in range(NBUF):
    wait_write(base + b, b)


def _embt(xt1, token_table, pos_table):
  mesh = plsc.VectorSubcoreMesh(core_axis_name="c", subcore_axis_name="s")
  scratch = [
      pltpu.VMEM((NIT * CB,), jnp.int32),
      pltpu.VMEM((L, D), jnp.float32),
      pltpu.VMEM((NBUF, CB, D), jnp.float32),
      pltpu.VMEM((NBUF, D, OBW), jnp.float32),
  ] + [pltpu.SemaphoreType.DMA] * (2 * NBUF)
  f = pl.kernel(
      _embt_body,
      out_type=jax.ShapeDtypeStruct((L, 8, NCB, 8, CB), jnp.float32),
      mesh=mesh,
      scratch_types=scratch,
      compiler_params=pltpu.CompilerParams(
          use_tc_tiling_on_sc=False, needs_layout_passes=False),
  )
  return f(xt1, token_table, pos_table)


def kernel(x, token_table, pos_table):
  b, l = x.shape
  d = token_table.shape[1]
  assert (b, l, d) == (B, L, D)
  xt1 = x.astype(jnp.int32).T.reshape(-1)
  out_phys = _embt(xt1, token_table, pos_table)
  return out_phys.transpose((2, 4, 0, 1, 3)).reshape(B, L, D)


# speedup vs baseline: 3.3096x; 1.1313x over previous
import jax
import jax.numpy as jnp
from jax import lax
from jax.experimental import pallas as pl
from jax.experimental.pallas import tpu as pltpu
from jax.experimental.pallas import tpu_sc as plsc

B = 4096
L = 200
D = 64
NC = 2
NS = 16
NW = NC * NS
CB = 128
NCB = B // CB
NIT = (L * NCB) // NW
NBUF = 5
PF = 3
OBW = 136


def _embt_body(xt_hbm, tok_hbm, pos_hbm, out_hbm, idx_v, pos_v, gbuf, obuf,
               *sems):
  gsem = sems[:NBUF]
  wsem = sems[NBUF:]
  wid = lax.axis_index("s") * NC + lax.axis_index("c")
  base_it = wid * NIT

  pltpu.sync_copy(xt_hbm.at[pl.ds(base_it * CB, NIT * CB)], idx_v)
  pltpu.sync_copy(pos_hbm, pos_v)

  def lc(i):
    g = base_it + i
    return lax.div(g, NCB), lax.rem(g, NCB)

  def issue_gather(i, b):
    pltpu.async_copy(tok_hbm.at[idx_v.at[pl.ds(i * CB, CB)]], gbuf.at[b],
                     gsem[b])

  def wait_gather(i, b):
    pltpu.make_async_copy(tok_hbm.at[idx_v.at[pl.ds(i * CB, CB)]], gbuf.at[b],
                          gsem[b]).wait()

  def ob_view(b, dh):
    return obuf.at[b, pl.ds(dh * 8, 8), pl.ds(0, CB)]

  def issue_write(i, b):
    l, c = lc(i)
    for dh in range(8):
      pltpu.async_copy(ob_view(b, dh), out_hbm.at[l, dh, c], wsem[b])

  def wait_write(i, b):
    l, c = lc(i)
    for dh in range(8):
      pltpu.make_async_copy(ob_view(b, dh), out_hbm.at[l, dh, c],
                            wsem[b]).wait()

  def transpose_add(i, b):
    l, _ = lc(i)
    gb = gbuf.at[b]
    ob = obuf.at[b]
    pos_rows = [pos_v[l, pl.ds(16 * j, 16)] for j in range(D // 16)]
    dvecs = [lax.iota(jnp.int32, 16) + (16 * j) for j in range(D // 16)]

    @plsc.parallel_loop(0, CB, unroll=2)
    def _(bb):
      bvec = jnp.full((16,), bb, jnp.int32)
      for j in range(D // 16):
        val = gb[bb, pl.ds(16 * j, 16)] + pos_rows[j]
        plsc.store_scatter(ob, [dvecs[j], bvec], val)

  def do_item(i, b, prefetch, drain):
    wait_gather(i, b)
    transpose_add(i, b)
    issue_write(i, b)
    if prefetch:
      tgt = i + PF
      bp = (b + PF) % NBUF
      if drain:
        wait_write(tgt - NBUF, bp)
      issue_gather(tgt, bp)

  for b in range(PF):
    issue_gather(jnp.int32(b), b)

  for b in range(NBUF):
    do_item(jnp.int32(b), b, prefetch=True, drain=(b >= NBUF - PF))

  def outer(step, _):
    for b in range(NBUF):
      i = step * NBUF + b
      do_item(i, b, prefetch=True, drain=True)
    return 0

  lax.fori_loop(1, NIT // NBUF - 1, outer, 0)

  base = jnp.int32(NIT - NBUF)
  for b in range(NBUF):
    do_item(base + b, b, prefetch=(b < NBUF - PF), drain=True)

  for b in range(NBUF):
    wait_write(base + b, b)


def _embt(xt1, token_table, pos_table):
  mesh = plsc.VectorSubcoreMesh(core_axis_name="c", subcore_axis_name="s")
  scratch = [
      pltpu.VMEM((NIT * CB,), jnp.int32),
      pltpu.VMEM((L, D), jnp.float32),
      pltpu.VMEM((NBUF, CB, D), jnp.float32),
      pltpu.VMEM((NBUF, D, OBW), jnp.float32),
  ] + [pltpu.SemaphoreType.DMA] * (2 * NBUF)
  f = pl.kernel(
      _embt_body,
      out_type=jax.ShapeDtypeStruct((L, 8, NCB, 8, CB), jnp.float32),
      mesh=mesh,
      scratch_types=scratch,
      compiler_params=pltpu.CompilerParams(
          use_tc_tiling_on_sc=False, needs_layout_passes=False),
  )
  return f(xt1, token_table, pos_table)


def kernel(x, token_table, pos_table):
  b, l = x.shape
  d = token_table.shape[1]
  assert (b, l, d) == (B, L, D)
  xt1 = x.astype(jnp.int32).T.reshape(-1)
  out_phys = _embt(xt1, token_table, pos_table)
  return out_phys.transpose((2, 4, 0, 1, 3)).reshape(B, L, D)


# speedup vs baseline: 3.3522x; 1.0129x over previous
import jax
import jax.numpy as jnp
from jax import lax
from jax.experimental import pallas as pl
from jax.experimental.pallas import tpu as pltpu
from jax.experimental.pallas import tpu_sc as plsc

B = 4096
L = 200
D = 64
NC = 2
NS = 16
NW = NC * NS
CB = 128
NCB = B // CB
NIT = (L * NCB) // NW
NBUF = 5
PF = 4
OBW = 136


def _embt_body(xt_hbm, tok_hbm, pos_hbm, out_hbm, idx_v, pos_v, gbuf, obuf,
               *sems):
  gsem = sems[:NBUF]
  wsem = sems[NBUF:]
  wid = lax.axis_index("s") * NC + lax.axis_index("c")
  base_it = wid * NIT

  pltpu.sync_copy(xt_hbm.at[pl.ds(base_it * CB, NIT * CB)], idx_v)
  pltpu.sync_copy(pos_hbm, pos_v)

  def lc(i):
    g = base_it + i
    return lax.div(g, NCB), lax.rem(g, NCB)

  def issue_gather(i, b):
    pltpu.async_copy(tok_hbm.at[idx_v.at[pl.ds(i * CB, CB)]], gbuf.at[b],
                     gsem[b])

  def wait_gather(i, b):
    pltpu.make_async_copy(tok_hbm.at[idx_v.at[pl.ds(i * CB, CB)]], gbuf.at[b],
                          gsem[b]).wait()

  def ob_view(b, dh):
    return obuf.at[b, pl.ds(dh * 8, 8), pl.ds(0, CB)]

  def issue_write(i, b):
    l, c = lc(i)
    for dh in range(8):
      pltpu.async_copy(ob_view(b, dh), out_hbm.at[l, dh, c], wsem[b])

  def wait_write(i, b):
    l, c = lc(i)
    for dh in range(8):
      pltpu.make_async_copy(ob_view(b, dh), out_hbm.at[l, dh, c],
                            wsem[b]).wait()

  def transpose_add(i, b):
    l, _ = lc(i)
    gb = gbuf.at[b]
    ob = obuf.at[b]
    pos_rows = [pos_v[l, pl.ds(16 * j, 16)] for j in range(D // 16)]
    dvecs = [lax.iota(jnp.int32, 16) + (16 * j) for j in range(D // 16)]

    @plsc.parallel_loop(0, CB, unroll=2)
    def _(bb):
      bvec = jnp.full((16,), bb, jnp.int32)
      for j in range(D // 16):
        val = gb[bb, pl.ds(16 * j, 16)] + pos_rows[j]
        plsc.store_scatter(ob, [dvecs[j], bvec], val)

  def do_item(i, b, prefetch, drain):
    wait_gather(i, b)
    transpose_add(i, b)
    issue_write(i, b)
    if prefetch:
      tgt = i + PF
      bp = (b + PF) % NBUF
      if drain:
        wait_write(tgt - NBUF, bp)
      issue_gather(tgt, bp)

  for b in range(PF):
    issue_gather(jnp.int32(b), b)

  for b in range(NBUF):
    do_item(jnp.int32(b), b, prefetch=True, drain=(b >= NBUF - PF))

  def outer(step, _):
    for b in range(NBUF):
      i = step * NBUF + b
      do_item(i, b, prefetch=True, drain=True)
    return 0

  lax.fori_loop(1, NIT // NBUF - 1, outer, 0)

  base = jnp.int32(NIT - NBUF)
  for b in range(NBUF):
    do_item(base + b, b, prefetch=(b < NBUF - PF), drain=True)

  for b in range(NBUF):
    wait_write(base + b, b)


def _embt(xt1, token_table, pos_table):
  mesh = plsc.VectorSubcoreMesh(core_axis_name="c", subcore_axis_name="s")
  scratch = [
      pltpu.VMEM((NIT * CB,), jnp.int32),
      pltpu.VMEM((L, D), jnp.float32),
      pltpu.VMEM((NBUF, CB, D), jnp.float32),
      pltpu.VMEM((NBUF, D, OBW), jnp.float32),
  ] + [pltpu.SemaphoreType.DMA] * (2 * NBUF)
  f = pl.kernel(
      _embt_body,
      out_type=jax.ShapeDtypeStruct((L, 8, NCB, 8, CB), jnp.float32),
      mesh=mesh,
      scratch_types=scratch,
      compiler_params=pltpu.CompilerParams(
          use_tc_tiling_on_sc=False, needs_layout_passes=False),
  )
  return f(xt1, token_table, pos_table)


def kernel(x, token_table, pos_table):
  b, l = x.shape
  d = token_table.shape[1]
  assert (b, l, d) == (B, L, D)
  xt1 = x.astype(jnp.int32).T.reshape(-1)
  out_phys = _embt(xt1, token_table, pos_table)
  return out_phys.transpose((2, 4, 0, 1, 3)).reshape(B, L, D)
